# baseline (speedup 1.0000x reference)
import functools

import jax
import jax.numpy as jnp
from jax import lax
from jax.experimental import pallas as pl
from jax.experimental.pallas import tpu as pltpu
from jax.experimental.pallas import tpu_sc as plsc

B = 1024
D = 128
K = 64
N = 10000
E = 320000
L = 10

NC = 2
NS = 16
NW = NC * NS

G = 128
EP = 327680
EW = EP // NW
NGRP = EW // G
HW = NGRP // 2
PAIRS_H = HW // 2
NP = 10240
RPT = NP // NS
BPW = B // NW

R = 1280
GRID = NP // R

_f32 = jnp.float32
_i32 = jnp.int32


def _mesh():
    return plsc.VectorSubcoreMesh(
        core_axis_name="c", subcore_axis_name="s",
        num_cores=NC, num_subcores=NS)


@functools.cache
def _get_edge_agg():
    @functools.partial(
        pl.kernel,
        mesh=_mesh(),
        out_type=jax.ShapeDtypeStruct((NC, NP, D), _f32),
        scratch_types=[
            pltpu.VMEM_SHARED((NP, D), _f32),
            pltpu.VMEM((HW, G), _i32),
            pltpu.VMEM((HW, G), _i32),
            pltpu.VMEM((HW, G), _f32),
            pltpu.VMEM((G, D), _f32),
            pltpu.VMEM((G, D), _f32),
            pltpu.SemaphoreType.DMA,
            pltpu.SemaphoreType.DMA,
            pltpu.SemaphoreType.DMA,
            pltpu.SemaphoreType.DMA,
        ],
    )
    def _edge_agg(m_hbm, src_hbm, dst_hbm, attr_hbm, part_hbm,
                  acc, srcb, dstb, attrb, rows0, rows1,
                  gsem0, gsem1, ssem0, ssem1):
        c = lax.axis_index("c")
        s = lax.axis_index("s")
        w = c * NS + s

        def _zr(i, carry):
            for k in range(D // 16):
                rows0[i, pl.ds(k * 16, 16)] = jnp.zeros((16,), _f32)
            return carry
        lax.fori_loop(0, G, _zr, 0)
        for j in range(RPT // G):
            pltpu.sync_copy(rows0, acc.at[pl.ds(s * RPT + j * G, G)])

        plsc.subcore_barrier()

        def _scale(rows, g):
            def _scale16(t, inner):
                avec = attrb[g, pl.ds(t * 16, 16)]
                for j in range(16):
                    a = avec[j]
                    r = t * 16 + j
                    for k in range(D // 16):
                        rows[r, pl.ds(k * 16, 16)] = (
                            rows[r, pl.ds(k * 16, 16)] * a)
                return inner
            lax.fori_loop(0, G // 16, _scale16, 0)

        for half in range(2):
            pltpu.sync_copy(src_hbm.at[w, pl.ds(half * HW, HW)], srcb)
            pltpu.sync_copy(dst_hbm.at[w, pl.ds(half * HW, HW)], dstb)
            pltpu.sync_copy(attr_hbm.at[w, pl.ds(half * HW, HW)], attrb)
            pltpu.async_copy(m_hbm.at[srcb.at[0]], rows0, gsem0)

            def _pair(p, carry):
                g0 = 2 * p
                g1 = 2 * p + 1
                pltpu.make_async_copy(
                    m_hbm.at[srcb.at[g0]], rows0, gsem0).wait()

                @pl.when(p > 0)
                def _():
                    pltpu.make_async_copy(
                        rows1, acc.at[dstb.at[g0 - 1]], ssem1).wait()
                pltpu.async_copy(m_hbm.at[srcb.at[g1]], rows1, gsem1)
                _scale(rows0, g0)
                pltpu.async_copy(rows0, acc.at[dstb.at[g0]], ssem0, add=True)
                pltpu.make_async_copy(
                    m_hbm.at[srcb.at[g1]], rows1, gsem1).wait()
                pltpu.make_async_copy(rows0, acc.at[dstb.at[g0]], ssem0).wait()

                @pl.when(p < PAIRS_H - 1)
                def _():
                    pltpu.async_copy(m_hbm.at[srcb.at[g1 + 1]], rows0, gsem0)
                _scale(rows1, g1)
                pltpu.async_copy(rows1, acc.at[dstb.at[g1]], ssem1, add=True)
                return carry
            lax.fori_loop(0, PAIRS_H, _pair, 0)
            pltpu.make_async_copy(rows1, acc.at[dstb.at[HW - 1]], ssem1).wait()

        plsc.subcore_barrier()
        pltpu.sync_copy(acc.at[pl.ds(s * RPT, RPT)],
                        part_hbm.at[c, pl.ds(s * RPT, RPT)])
    return _edge_agg


@functools.cache
def _get_emb_gather():
    @functools.partial(
        pl.kernel,
        mesh=_mesh(),
        out_type=(jax.ShapeDtypeStruct((B, D), _f32),
                  jax.ShapeDtypeStruct((B, D), _f32)),
        scratch_types=[
            pltpu.VMEM((BPW,), _i32),
            pltpu.VMEM((BPW, D), _f32),
            pltpu.SemaphoreType.DMA,
        ],
    )
    def _emb_gather(ue_hbm, ve_hbm, bu_hbm, bv_hbm, gu_hbm, gv_hbm,
                    idx, buf, sem):
        c = lax.axis_index("c")
        s = lax.axis_index("s")
        base = (c * NS + s) * BPW
        pltpu.sync_copy(bu_hbm.at[pl.ds(base, BPW)], idx)
        pltpu.async_copy(ue_hbm.at[idx], buf, sem).wait()
        pltpu.sync_copy(buf, gu_hbm.at[pl.ds(base, BPW)])
        pltpu.sync_copy(bv_hbm.at[pl.ds(base, BPW)], idx)
        pltpu.async_copy(ve_hbm.at[idx], buf, sem).wait()
        pltpu.sync_copy(buf, gv_hbm.at[pl.ds(base, BPW)])
    return _emb_gather


def _prep_body(x_ref, lab_ref, det_ref, wm_ref, ws_ref, b_ref, m_ref, s_ref):
    lab = lab_ref[...]
    iota = lax.broadcasted_iota(_i32, (R, 16), 1)
    onehot = (lab == iota).astype(_f32)
    de = jnp.dot(onehot, det_ref[...], preferred_element_type=_f32)
    x = x_ref[...]
    wm = wm_ref[...]
    ws = ws_ref[...]
    m_ref[...] = (jnp.dot(x, wm[:D], preferred_element_type=_f32)
                  + jnp.dot(de, wm[D:], preferred_element_type=_f32))
    s_ref[...] = (jnp.dot(x, ws[:D], preferred_element_type=_f32)
                  + jnp.dot(de, ws[D:], preferred_element_type=_f32)
                  + b_ref[...])


_prep = pl.pallas_call(
    _prep_body,
    grid=(GRID,),
    in_specs=[
        pl.BlockSpec((R, D), lambda i: (i, 0)),
        pl.BlockSpec((R, 1), lambda i: (i, 0)),
        pl.BlockSpec((16, D), lambda i: (0, 0)),
        pl.BlockSpec((2 * D, D), lambda i: (0, 0)),
        pl.BlockSpec((2 * D, D), lambda i: (0, 0)),
        pl.BlockSpec((1, D), lambda i: (0, 0)),
    ],
    out_specs=[pl.BlockSpec((R, D), lambda i: (i, 0))] * 2,
    out_shape=[jax.ShapeDtypeStruct((NP, D), _f32)] * 2,
)


def _comb_body(p0_ref, p1_ref, sin_ref, wm_ref, ws_ref, b_ref, m_ref, s_ref):
    h = jnp.maximum(p0_ref[...] + p1_ref[...] + sin_ref[...], 0.0)
    m_ref[...] = jnp.dot(h, wm_ref[...], preferred_element_type=_f32)
    s_ref[...] = jnp.dot(h, ws_ref[...], preferred_element_type=_f32) + b_ref[...]


_comb = pl.pallas_call(
    _comb_body,
    grid=(GRID,),
    in_specs=[
        pl.BlockSpec((R, D), lambda i: (i, 0)),
        pl.BlockSpec((R, D), lambda i: (i, 0)),
        pl.BlockSpec((R, D), lambda i: (i, 0)),
        pl.BlockSpec((D, D), lambda i: (0, 0)),
        pl.BlockSpec((D, D), lambda i: (0, 0)),
        pl.BlockSpec((1, D), lambda i: (0, 0)),
    ],
    out_specs=[pl.BlockSpec((R, D), lambda i: (i, 0))] * 2,
    out_shape=[jax.ShapeDtypeStruct((NP, D), _f32)] * 2,
)


def _pool_body(q0_ref, q1_ref, sin_ref, sb_ref, sums_ref, cnt_ref):
    h = jnp.maximum(q0_ref[...] + q1_ref[...] + sin_ref[...], 0.0)
    sb = sb_ref[...]
    iota = lax.broadcasted_iota(_i32, (R, B), 1)
    onehot = (sb == iota).astype(_f32)
    psum = lax.dot_general(onehot, h, (((0,), (0,)), ((), ())),
                           preferred_element_type=_f32)
    ones = jnp.ones((R, 1), _f32)
    pcnt = lax.dot_general(onehot, ones, (((0,), (0,)), ((), ())),
                           preferred_element_type=_f32)

    @pl.when(pl.program_id(0) == 0)
    def _():
        sums_ref[...] = jnp.zeros_like(sums_ref)
        cnt_ref[...] = jnp.zeros_like(cnt_ref)

    sums_ref[...] += psum
    cnt_ref[...] += pcnt


_pool = pl.pallas_call(
    _pool_body,
    grid=(GRID,),
    in_specs=[
        pl.BlockSpec((R, D), lambda i: (i, 0)),
        pl.BlockSpec((R, D), lambda i: (i, 0)),
        pl.BlockSpec((R, D), lambda i: (i, 0)),
        pl.BlockSpec((R, 1), lambda i: (i, 0)),
    ],
    out_specs=[pl.BlockSpec((B, D), lambda i: (0, 0)),
               pl.BlockSpec((B, 1), lambda i: (0, 0))],
    out_shape=[jax.ShapeDtypeStruct((B, D), _f32),
               jax.ShapeDtypeStruct((B, 1), _f32)],
)


def _softmax(x):
    z = x - jnp.max(x, axis=1, keepdims=True)
    e = jnp.exp(z)
    return e / jnp.sum(e, axis=1, keepdims=True)


def _head_body(sums_ref, cnt_ref, gu_ref, gv_ref, cu_ref, cv_ref, rf_ref,
               wp_ref, bp_ref, wm1_ref, bm1_ref, wm2_ref, bm2_ref,
               pred_ref, pu_ref, hsub_ref):
    cnt = jnp.maximum(cnt_ref[...], 1.0)
    pooled = sums_ref[...] / cnt
    hsub = jnp.tanh(jnp.dot(pooled, wp_ref[...], preferred_element_type=_f32)
                    + bp_ref[...])
    gu = gu_ref[...]
    gv = gv_ref[...]
    pu = _softmax(lax.dot_general(gu, cu_ref[...], (((1,), (1,)), ((), ())),
                                  preferred_element_type=_f32))
    pv = _softmax(lax.dot_general(gv, cv_ref[...], (((1,), (1,)), ((), ())),
                                  preferred_element_type=_f32))
    wm1 = wm1_ref[...]
    logits = (jnp.dot(hsub, wm1[0:D], preferred_element_type=_f32)
              + jnp.dot(gu, wm1[D:2 * D], preferred_element_type=_f32)
              + jnp.dot(gv, wm1[2 * D:3 * D], preferred_element_type=_f32)
              + jnp.dot(pu, wm1[3 * D:3 * D + K], preferred_element_type=_f32)
              + jnp.dot(pv, wm1[3 * D + K:3 * D + 2 * K],
                        preferred_element_type=_f32)
              + jnp.dot(rf_ref[...], wm1[3 * D + 2 * K:],
                        preferred_element_type=_f32)
              + bm1_ref[...])
    act = jnp.maximum(logits, 0.0)
    o = jnp.dot(act, wm2_ref[...], preferred_element_type=_f32) + bm2_ref[...]
    pred_ref[...] = 1.0 / (1.0 + jnp.exp(-o))
    pu_ref[...] = pu
    hsub_ref[...] = hsub


_head = pl.pallas_call(
    _head_body,
    out_shape=[jax.ShapeDtypeStruct((B, 1), _f32),
               jax.ShapeDtypeStruct((B, K), _f32),
               jax.ShapeDtypeStruct((B, D), _f32)],
)


def kernel(batch_u, batch_v, global_u_emb, global_v_emb, sub_x,
           sub_edge_index, sub_edge_attr, sub_batch, sub_dist_labels,
           rand_feat, c_u, c_v, dist_label_emb, W_msg0, W_self0, b0,
           W_msg1, W_self1, b1, W_pool, b_pool, W_m1, b_m1, W_m2, b_m2):
    bu = batch_u.astype(_i32)
    bv = batch_v.astype(_i32)
    src = sub_edge_index[0].astype(_i32)
    dst = sub_edge_index[1].astype(_i32)
    attr = sub_edge_attr.reshape(E).astype(_f32)

    pad_e = EP - E
    src_p = jnp.concatenate([src, jnp.zeros((pad_e,), _i32)]).reshape(NW, NGRP, G)
    dst_p = jnp.concatenate([dst, jnp.zeros((pad_e,), _i32)]).reshape(NW, NGRP, G)
    attr_p = jnp.concatenate([attr, jnp.zeros((pad_e,), _f32)]).reshape(NW, NGRP, G)

    pad_n = NP - N
    x_p = jnp.concatenate([sub_x, jnp.zeros((pad_n, D), _f32)], axis=0)
    lab_p = jnp.concatenate([sub_dist_labels.astype(_i32),
                             jnp.zeros((pad_n,), _i32)]).reshape(NP, 1)
    sb_p = jnp.concatenate([sub_batch.astype(_i32),
                            jnp.full((pad_n,), B, _i32)]).reshape(NP, 1)
    det_p = jnp.concatenate([dist_label_emb, jnp.zeros((16 - L, D), _f32)],
                            axis=0)

    m0, s0 = _prep(x_p, lab_p, det_p, W_msg0, W_self0, b0.reshape(1, D))
    gu, gv = _get_emb_gather()(global_u_emb, global_v_emb, bu, bv)
    edge_agg = _get_edge_agg()
    part0 = edge_agg(m0, src_p, dst_p, attr_p)
    m1, s1 = _comb(part0[0], part0[1], s0, W_msg1, W_self1, b1.reshape(1, D))
    part1 = edge_agg(m1, src_p, dst_p, attr_p)
    sums, cnt = _pool(part1[0], part1[1], s1, sb_p)
    pred, pu, hsub = _head(sums, cnt, gu, gv, c_u, c_v, rand_feat,
                           W_pool, b_pool.reshape(1, D),
                           W_m1, b_m1.reshape(1, K),
                           W_m2, b_m2.reshape(1, 1))
    return pred.reshape(B), pu, hsub, pu


# speedup vs baseline: 1.0377x; 1.0377x over previous
import functools

import jax
import jax.numpy as jnp
from jax import lax
from jax.experimental import pallas as pl
from jax.experimental.pallas import tpu as pltpu
from jax.experimental.pallas import tpu_sc as plsc

B = 1024
D = 128
K = 64
N = 10000
E = 320000
L = 10

NC = 2
NS = 16
NW = NC * NS

G = 128
EP = 327680
TOTG = EP // G
Q0 = 112
Q1 = 48
W = 16
PAIRS_H = W // 2
NP = 10240
RPT = NP // NS
BPW = B // NW

R = 1280
GRID = NP // R

_f32 = jnp.float32
_i32 = jnp.int32


def _mesh():
    return plsc.VectorSubcoreMesh(
        core_axis_name="c", subcore_axis_name="s",
        num_cores=NC, num_subcores=NS)


@functools.cache
def _get_edge_agg():
    @functools.partial(
        pl.kernel,
        mesh=_mesh(),
        out_type=jax.ShapeDtypeStruct((NC, NP, D), _f32),
        scratch_types=[
            pltpu.VMEM_SHARED((NP, D), _f32),
            pltpu.VMEM((W, G), _i32),
            pltpu.VMEM((W, G), _i32),
            pltpu.VMEM((W, G), _f32),
            pltpu.VMEM((G, D), _f32),
            pltpu.VMEM((G, D), _f32),
            pltpu.SemaphoreType.DMA,
            pltpu.SemaphoreType.DMA,
            pltpu.SemaphoreType.DMA,
            pltpu.SemaphoreType.DMA,
        ],
    )
    def _edge_agg(m_hbm, src_hbm, dst_hbm, attr_hbm, part_hbm,
                  acc, srcb, dstb, attrb, rows0, rows1,
                  gsem0, gsem1, ssem0, ssem1):
        c = lax.axis_index("c")
        s = lax.axis_index("s")
        gbase = jnp.where(c == 0, s * Q0, NS * Q0 + s * Q1)
        nwin = jnp.where(c == 0, Q0 // W, Q1 // W)

        def _zr(i, carry):
            for k in range(D // 16):
                rows0[i, pl.ds(k * 16, 16)] = jnp.zeros((16,), _f32)
            return carry
        lax.fori_loop(0, G, _zr, 0)
        for j in range(RPT // G):
            pltpu.sync_copy(rows0, acc.at[pl.ds(s * RPT + j * G, G)])

        plsc.subcore_barrier()

        def _scale(rows, g):
            def _scale16(t, inner):
                avec = attrb[g, pl.ds(t * 16, 16)]
                for j in range(16):
                    a = avec[j]
                    r = t * 16 + j
                    for k in range(D // 16):
                        rows[r, pl.ds(k * 16, 16)] = (
                            rows[r, pl.ds(k * 16, 16)] * a)
                return inner
            lax.fori_loop(0, G // 16, _scale16, 0)

        def _window(win, wcarry):
            wstart = pl.multiple_of(gbase + win * W, W)
            pltpu.sync_copy(src_hbm.at[pl.ds(wstart, W)], srcb)
            pltpu.sync_copy(dst_hbm.at[pl.ds(wstart, W)], dstb)
            pltpu.sync_copy(attr_hbm.at[pl.ds(wstart, W)], attrb)
            pltpu.async_copy(m_hbm.at[srcb.at[0]], rows0, gsem0)

            def _pair(p, carry):
                g0 = 2 * p
                g1 = 2 * p + 1
                pltpu.make_async_copy(
                    m_hbm.at[srcb.at[g0]], rows0, gsem0).wait()

                @pl.when(p > 0)
                def _():
                    pltpu.make_async_copy(
                        rows1, acc.at[dstb.at[g0 - 1]], ssem1).wait()
                pltpu.async_copy(m_hbm.at[srcb.at[g1]], rows1, gsem1)
                _scale(rows0, g0)
                pltpu.async_copy(rows0, acc.at[dstb.at[g0]], ssem0, add=True)
                pltpu.make_async_copy(
                    m_hbm.at[srcb.at[g1]], rows1, gsem1).wait()
                pltpu.make_async_copy(rows0, acc.at[dstb.at[g0]], ssem0).wait()

                @pl.when(p < PAIRS_H - 1)
                def _():
                    pltpu.async_copy(m_hbm.at[srcb.at[g1 + 1]], rows0, gsem0)
                _scale(rows1, g1)
                pltpu.async_copy(rows1, acc.at[dstb.at[g1]], ssem1, add=True)
                return carry
            lax.fori_loop(0, PAIRS_H, _pair, 0)
            pltpu.make_async_copy(rows1, acc.at[dstb.at[W - 1]], ssem1).wait()
            return wcarry
        lax.fori_loop(0, nwin, _window, 0)

        plsc.subcore_barrier()
        pltpu.sync_copy(acc.at[pl.ds(s * RPT, RPT)],
                        part_hbm.at[c, pl.ds(s * RPT, RPT)])
    return _edge_agg


@functools.cache
def _get_emb_gather():
    @functools.partial(
        pl.kernel,
        mesh=_mesh(),
        out_type=(jax.ShapeDtypeStruct((B, D), _f32),
                  jax.ShapeDtypeStruct((B, D), _f32)),
        scratch_types=[
            pltpu.VMEM((BPW,), _i32),
            pltpu.VMEM((BPW, D), _f32),
            pltpu.SemaphoreType.DMA,
        ],
    )
    def _emb_gather(ue_hbm, ve_hbm, bu_hbm, bv_hbm, gu_hbm, gv_hbm,
                    idx, buf, sem):
        c = lax.axis_index("c")
        s = lax.axis_index("s")
        base = (c * NS + s) * BPW
        pltpu.sync_copy(bu_hbm.at[pl.ds(base, BPW)], idx)
        pltpu.async_copy(ue_hbm.at[idx], buf, sem).wait()
        pltpu.sync_copy(buf, gu_hbm.at[pl.ds(base, BPW)])
        pltpu.sync_copy(bv_hbm.at[pl.ds(base, BPW)], idx)
        pltpu.async_copy(ve_hbm.at[idx], buf, sem).wait()
        pltpu.sync_copy(buf, gv_hbm.at[pl.ds(base, BPW)])
    return _emb_gather


def _prep_body(x_ref, lab_ref, det_ref, wm_ref, ws_ref, b_ref, m_ref, s_ref):
    lab = lab_ref[...]
    iota = lax.broadcasted_iota(_i32, (R, 16), 1)
    onehot = (lab == iota).astype(_f32)
    de = jnp.dot(onehot, det_ref[...], preferred_element_type=_f32)
    x = x_ref[...]
    wm = wm_ref[...]
    ws = ws_ref[...]
    m_ref[...] = (jnp.dot(x, wm[:D], preferred_element_type=_f32)
                  + jnp.dot(de, wm[D:], preferred_element_type=_f32))
    s_ref[...] = (jnp.dot(x, ws[:D], preferred_element_type=_f32)
                  + jnp.dot(de, ws[D:], preferred_element_type=_f32)
                  + b_ref[...])


_prep = pl.pallas_call(
    _prep_body,
    grid=(GRID,),
    in_specs=[
        pl.BlockSpec((R, D), lambda i: (i, 0)),
        pl.BlockSpec((R, 1), lambda i: (i, 0)),
        pl.BlockSpec((16, D), lambda i: (0, 0)),
        pl.BlockSpec((2 * D, D), lambda i: (0, 0)),
        pl.BlockSpec((2 * D, D), lambda i: (0, 0)),
        pl.BlockSpec((1, D), lambda i: (0, 0)),
    ],
    out_specs=[pl.BlockSpec((R, D), lambda i: (i, 0))] * 2,
    out_shape=[jax.ShapeDtypeStruct((NP, D), _f32)] * 2,
)


def _comb_body(p0_ref, p1_ref, sin_ref, wm_ref, ws_ref, b_ref, m_ref, s_ref):
    h = jnp.maximum(p0_ref[...] + p1_ref[...] + sin_ref[...], 0.0)
    m_ref[...] = jnp.dot(h, wm_ref[...], preferred_element_type=_f32)
    s_ref[...] = jnp.dot(h, ws_ref[...], preferred_element_type=_f32) + b_ref[...]


_comb = pl.pallas_call(
    _comb_body,
    grid=(GRID,),
    in_specs=[
        pl.BlockSpec((R, D), lambda i: (i, 0)),
        pl.BlockSpec((R, D), lambda i: (i, 0)),
        pl.BlockSpec((R, D), lambda i: (i, 0)),
        pl.BlockSpec((D, D), lambda i: (0, 0)),
        pl.BlockSpec((D, D), lambda i: (0, 0)),
        pl.BlockSpec((1, D), lambda i: (0, 0)),
    ],
    out_specs=[pl.BlockSpec((R, D), lambda i: (i, 0))] * 2,
    out_shape=[jax.ShapeDtypeStruct((NP, D), _f32)] * 2,
)


def _pool_body(q0_ref, q1_ref, sin_ref, sb_ref, sums_ref, cnt_ref):
    h = jnp.maximum(q0_ref[...] + q1_ref[...] + sin_ref[...], 0.0)
    sb = sb_ref[...]
    iota = lax.broadcasted_iota(_i32, (R, B), 1)
    onehot = (sb == iota).astype(_f32)
    psum = lax.dot_general(onehot, h, (((0,), (0,)), ((), ())),
                           preferred_element_type=_f32)
    ones = jnp.ones((R, 1), _f32)
    pcnt = lax.dot_general(onehot, ones, (((0,), (0,)), ((), ())),
                           preferred_element_type=_f32)

    @pl.when(pl.program_id(0) == 0)
    def _():
        sums_ref[...] = jnp.zeros_like(sums_ref)
        cnt_ref[...] = jnp.zeros_like(cnt_ref)

    sums_ref[...] += psum
    cnt_ref[...] += pcnt


_pool = pl.pallas_call(
    _pool_body,
    grid=(GRID,),
    in_specs=[
        pl.BlockSpec((R, D), lambda i: (i, 0)),
        pl.BlockSpec((R, D), lambda i: (i, 0)),
        pl.BlockSpec((R, D), lambda i: (i, 0)),
        pl.BlockSpec((R, 1), lambda i: (i, 0)),
    ],
    out_specs=[pl.BlockSpec((B, D), lambda i: (0, 0)),
               pl.BlockSpec((B, 1), lambda i: (0, 0))],
    out_shape=[jax.ShapeDtypeStruct((B, D), _f32),
               jax.ShapeDtypeStruct((B, 1), _f32)],
)


def _softmax(x):
    z = x - jnp.max(x, axis=1, keepdims=True)
    e = jnp.exp(z)
    return e / jnp.sum(e, axis=1, keepdims=True)


def _head_body(sums_ref, cnt_ref, gu_ref, gv_ref, cu_ref, cv_ref, rf_ref,
               wp_ref, bp_ref, wm1_ref, bm1_ref, wm2_ref, bm2_ref,
               pred_ref, pu_ref, hsub_ref):
    cnt = jnp.maximum(cnt_ref[...], 1.0)
    pooled = sums_ref[...] / cnt
    hsub = jnp.tanh(jnp.dot(pooled, wp_ref[...], preferred_element_type=_f32)
                    + bp_ref[...])
    gu = gu_ref[...]
    gv = gv_ref[...]
    pu = _softmax(lax.dot_general(gu, cu_ref[...], (((1,), (1,)), ((), ())),
                                  preferred_element_type=_f32))
    pv = _softmax(lax.dot_general(gv, cv_ref[...], (((1,), (1,)), ((), ())),
                                  preferred_element_type=_f32))
    wm1 = wm1_ref[...]
    logits = (jnp.dot(hsub, wm1[0:D], preferred_element_type=_f32)
              + jnp.dot(gu, wm1[D:2 * D], preferred_element_type=_f32)
              + jnp.dot(gv, wm1[2 * D:3 * D], preferred_element_type=_f32)
              + jnp.dot(pu, wm1[3 * D:3 * D + K], preferred_element_type=_f32)
              + jnp.dot(pv, wm1[3 * D + K:3 * D + 2 * K],
                        preferred_element_type=_f32)
              + jnp.dot(rf_ref[...], wm1[3 * D + 2 * K:],
                        preferred_element_type=_f32)
              + bm1_ref[...])
    act = jnp.maximum(logits, 0.0)
    o = jnp.dot(act, wm2_ref[...], preferred_element_type=_f32) + bm2_ref[...]
    pred_ref[...] = 1.0 / (1.0 + jnp.exp(-o))
    pu_ref[...] = pu
    hsub_ref[...] = hsub


_head = pl.pallas_call(
    _head_body,
    out_shape=[jax.ShapeDtypeStruct((B, 1), _f32),
               jax.ShapeDtypeStruct((B, K), _f32),
               jax.ShapeDtypeStruct((B, D), _f32)],
)


def kernel(batch_u, batch_v, global_u_emb, global_v_emb, sub_x,
           sub_edge_index, sub_edge_attr, sub_batch, sub_dist_labels,
           rand_feat, c_u, c_v, dist_label_emb, W_msg0, W_self0, b0,
           W_msg1, W_self1, b1, W_pool, b_pool, W_m1, b_m1, W_m2, b_m2):
    bu = batch_u.astype(_i32)
    bv = batch_v.astype(_i32)
    src = sub_edge_index[0].astype(_i32)
    dst = sub_edge_index[1].astype(_i32)
    attr = sub_edge_attr.reshape(E).astype(_f32)

    pad_e = EP - E
    src_p = jnp.concatenate([src, jnp.zeros((pad_e,), _i32)]).reshape(TOTG, G)
    dst_p = jnp.concatenate([dst, jnp.zeros((pad_e,), _i32)]).reshape(TOTG, G)
    attr_p = jnp.concatenate([attr, jnp.zeros((pad_e,), _f32)]).reshape(TOTG, G)

    pad_n = NP - N
    x_p = jnp.concatenate([sub_x, jnp.zeros((pad_n, D), _f32)], axis=0)
    lab_p = jnp.concatenate([sub_dist_labels.astype(_i32),
                             jnp.zeros((pad_n,), _i32)]).reshape(NP, 1)
    sb_p = jnp.concatenate([sub_batch.astype(_i32),
                            jnp.full((pad_n,), B, _i32)]).reshape(NP, 1)
    det_p = jnp.concatenate([dist_label_emb, jnp.zeros((16 - L, D), _f32)],
                            axis=0)

    m0, s0 = _prep(x_p, lab_p, det_p, W_msg0, W_self0, b0.reshape(1, D))
    gu, gv = _get_emb_gather()(global_u_emb, global_v_emb, bu, bv)
    edge_agg = _get_edge_agg()
    part0 = edge_agg(m0, src_p, dst_p, attr_p)
    m1, s1 = _comb(part0[0], part0[1], s0, W_msg1, W_self1, b1.reshape(1, D))
    part1 = edge_agg(m1, src_p, dst_p, attr_p)
    sums, cnt = _pool(part1[0], part1[1], s1, sb_p)
    pred, pu, hsub = _head(sums, cnt, gu, gv, c_u, c_v, rand_feat,
                           W_pool, b_pool.reshape(1, D),
                           W_m1, b_m1.reshape(1, K),
                           W_m2, b_m2.reshape(1, 1))
    return pred.reshape(B), pu, hsub, pu


# speedup vs baseline: 2.6558x; 2.5593x over previous
import functools

import jax
import jax.numpy as jnp
from jax import lax
from jax.experimental import pallas as pl
from jax.experimental.pallas import tpu as pltpu
from jax.experimental.pallas import tpu_sc as plsc

B = 1024
D = 128
K = 64
N = 10000
E = 320000
L = 10

NC = 2
NS = 16
NW = NC * NS

G = 128
EP = 327680
TOTG = EP // G
Q0 = 80
Q1 = 80
W = 16
PAIRS_H = W // 2
NP = 10240
RPT = NP // NS
BPW = B // NW

R = 1280
GRID = NP // R

_f32 = jnp.float32
_i32 = jnp.int32


def _mesh():
    return plsc.VectorSubcoreMesh(
        core_axis_name="c", subcore_axis_name="s",
        num_cores=NC, num_subcores=NS)


@functools.cache
def _get_edge_agg():
    @functools.partial(
        pl.kernel,
        mesh=_mesh(),
        out_type=jax.ShapeDtypeStruct((NC, NP, D), _f32),
        scratch_types=[
            pltpu.VMEM_SHARED((NP, D), _f32),
            pltpu.VMEM((W, G), _i32),
            pltpu.VMEM((W, G), _i32),
            pltpu.VMEM((W, G), _f32),
            pltpu.VMEM((G, D), _f32),
            pltpu.VMEM((G, D), _f32),
            pltpu.SemaphoreType.DMA,
            pltpu.SemaphoreType.DMA,
            pltpu.SemaphoreType.DMA,
            pltpu.SemaphoreType.DMA,
        ],
    )
    def _edge_agg(m_hbm, src_hbm, dst_hbm, attr_hbm, part_hbm,
                  acc, srcb, dstb, attrb, rows0, rows1,
                  gsem0, gsem1, ssem0, ssem1):
        c = lax.axis_index("c")
        s = lax.axis_index("s")
        gbase = jnp.where(c == 0, s * Q0, NS * Q0 + s * Q1)
        nwin = jnp.where(c == 0, Q0 // W, Q1 // W)

        def _zr(i, carry):
            for k in range(D // 16):
                rows0[i, pl.ds(k * 16, 16)] = jnp.zeros((16,), _f32)
            return carry
        lax.fori_loop(0, G, _zr, 0)
        for j in range(RPT // G):
            pltpu.sync_copy(rows0, acc.at[pl.ds(s * RPT + j * G, G)])

        plsc.subcore_barrier()

        def _scale(rows, g):
            def _scale16(t, inner):
                avec = attrb[g, pl.ds(t * 16, 16)]
                for j in range(16):
                    a = avec[j]
                    r = t * 16 + j
                    for k in range(D // 16):
                        rows[r, pl.ds(k * 16, 16)] = (
                            rows[r, pl.ds(k * 16, 16)] * a)
                return inner
            lax.fori_loop(0, G // 16, _scale16, 0)

        def _window(win, wcarry):
            wstart = pl.multiple_of(gbase + win * W, W)
            pltpu.sync_copy(src_hbm.at[pl.ds(wstart, W)], srcb)
            pltpu.sync_copy(dst_hbm.at[pl.ds(wstart, W)], dstb)
            pltpu.sync_copy(attr_hbm.at[pl.ds(wstart, W)], attrb)
            pltpu.async_copy(m_hbm.at[srcb.at[0]], rows0, gsem0)

            def _pair(p, carry):
                g0 = 2 * p
                g1 = 2 * p + 1
                pltpu.make_async_copy(
                    m_hbm.at[srcb.at[g0]], rows0, gsem0).wait()

                @pl.when(p > 0)
                def _():
                    pltpu.make_async_copy(
                        rows1, acc.at[dstb.at[g0 - 1]], ssem1).wait()
                pltpu.async_copy(m_hbm.at[srcb.at[g1]], rows1, gsem1)
                _scale(rows0, g0)
                pltpu.async_copy(rows0, acc.at[dstb.at[g0]], ssem0, add=True)
                pltpu.make_async_copy(
                    m_hbm.at[srcb.at[g1]], rows1, gsem1).wait()
                pltpu.make_async_copy(rows0, acc.at[dstb.at[g0]], ssem0).wait()

                @pl.when(p < PAIRS_H - 1)
                def _():
                    pltpu.async_copy(m_hbm.at[srcb.at[g1 + 1]], rows0, gsem0)
                _scale(rows1, g1)
                pltpu.async_copy(rows1, acc.at[dstb.at[g1]], ssem1, add=True)
                return carry
            lax.fori_loop(0, PAIRS_H, _pair, 0)
            pltpu.make_async_copy(rows1, acc.at[dstb.at[W - 1]], ssem1).wait()
            return wcarry
        lax.fori_loop(0, nwin, _window, 0)

        plsc.subcore_barrier()
        pltpu.sync_copy(acc.at[pl.ds(s * RPT, RPT)],
                        part_hbm.at[c, pl.ds(s * RPT, RPT)])
    return _edge_agg


@functools.cache
def _get_emb_gather():
    @functools.partial(
        pl.kernel,
        mesh=_mesh(),
        out_type=(jax.ShapeDtypeStruct((B, D), _f32),
                  jax.ShapeDtypeStruct((B, D), _f32)),
        scratch_types=[
            pltpu.VMEM((BPW,), _i32),
            pltpu.VMEM((BPW, D), _f32),
            pltpu.SemaphoreType.DMA,
        ],
    )
    def _emb_gather(ue_hbm, ve_hbm, bu_hbm, bv_hbm, gu_hbm, gv_hbm,
                    idx, buf, sem):
        c = lax.axis_index("c")
        s = lax.axis_index("s")
        base = (c * NS + s) * BPW
        pltpu.sync_copy(bu_hbm.at[pl.ds(base, BPW)], idx)
        pltpu.async_copy(ue_hbm.at[idx], buf, sem).wait()
        pltpu.sync_copy(buf, gu_hbm.at[pl.ds(base, BPW)])
        pltpu.sync_copy(bv_hbm.at[pl.ds(base, BPW)], idx)
        pltpu.async_copy(ve_hbm.at[idx], buf, sem).wait()
        pltpu.sync_copy(buf, gv_hbm.at[pl.ds(base, BPW)])
    return _emb_gather


def _prep_body(x_ref, lab_ref, det_ref, wm_ref, ws_ref, b_ref, m_ref, s_ref):
    lab = lab_ref[...]
    iota = lax.broadcasted_iota(_i32, (R, 16), 1)
    onehot = (lab == iota).astype(_f32)
    de = jnp.dot(onehot, det_ref[...], preferred_element_type=_f32)
    x = x_ref[...]
    wm = wm_ref[...]
    ws = ws_ref[...]
    m_ref[...] = (jnp.dot(x, wm[:D], preferred_element_type=_f32)
                  + jnp.dot(de, wm[D:], preferred_element_type=_f32))
    s_ref[...] = (jnp.dot(x, ws[:D], preferred_element_type=_f32)
                  + jnp.dot(de, ws[D:], preferred_element_type=_f32)
                  + b_ref[...])


_prep = pl.pallas_call(
    _prep_body,
    grid=(GRID,),
    in_specs=[
        pl.BlockSpec((R, D), lambda i: (i, 0)),
        pl.BlockSpec((R, 1), lambda i: (i, 0)),
        pl.BlockSpec((16, D), lambda i: (0, 0)),
        pl.BlockSpec((2 * D, D), lambda i: (0, 0)),
        pl.BlockSpec((2 * D, D), lambda i: (0, 0)),
        pl.BlockSpec((1, D), lambda i: (0, 0)),
    ],
    out_specs=[pl.BlockSpec((R, D), lambda i: (i, 0))] * 2,
    out_shape=[jax.ShapeDtypeStruct((NP, D), _f32)] * 2,
)


def _comb_body(p0_ref, p1_ref, sin_ref, wm_ref, ws_ref, b_ref, m_ref, s_ref):
    h = jnp.maximum(p0_ref[...] + p1_ref[...] + sin_ref[...], 0.0)
    m_ref[...] = jnp.dot(h, wm_ref[...], preferred_element_type=_f32)
    s_ref[...] = jnp.dot(h, ws_ref[...], preferred_element_type=_f32) + b_ref[...]


_comb = pl.pallas_call(
    _comb_body,
    grid=(GRID,),
    in_specs=[
        pl.BlockSpec((R, D), lambda i: (i, 0)),
        pl.BlockSpec((R, D), lambda i: (i, 0)),
        pl.BlockSpec((R, D), lambda i: (i, 0)),
        pl.BlockSpec((D, D), lambda i: (0, 0)),
        pl.BlockSpec((D, D), lambda i: (0, 0)),
        pl.BlockSpec((1, D), lambda i: (0, 0)),
    ],
    out_specs=[pl.BlockSpec((R, D), lambda i: (i, 0))] * 2,
    out_shape=[jax.ShapeDtypeStruct((NP, D), _f32)] * 2,
)


def _pool_body(q0_ref, q1_ref, sin_ref, sb_ref, sums_ref, cnt_ref):
    h = jnp.maximum(q0_ref[...] + q1_ref[...] + sin_ref[...], 0.0)
    sb = sb_ref[...]
    iota = lax.broadcasted_iota(_i32, (R, B), 1)
    onehot = (sb == iota).astype(_f32)
    psum = lax.dot_general(onehot, h, (((0,), (0,)), ((), ())),
                           preferred_element_type=_f32)
    ones = jnp.ones((R, 1), _f32)
    pcnt = lax.dot_general(onehot, ones, (((0,), (0,)), ((), ())),
                           preferred_element_type=_f32)

    @pl.when(pl.program_id(0) == 0)
    def _():
        sums_ref[...] = jnp.zeros_like(sums_ref)
        cnt_ref[...] = jnp.zeros_like(cnt_ref)

    sums_ref[...] += psum
    cnt_ref[...] += pcnt


_pool = pl.pallas_call(
    _pool_body,
    grid=(GRID,),
    in_specs=[
        pl.BlockSpec((R, D), lambda i: (i, 0)),
        pl.BlockSpec((R, D), lambda i: (i, 0)),
        pl.BlockSpec((R, D), lambda i: (i, 0)),
        pl.BlockSpec((R, 1), lambda i: (i, 0)),
    ],
    out_specs=[pl.BlockSpec((B, D), lambda i: (0, 0)),
               pl.BlockSpec((B, 1), lambda i: (0, 0))],
    out_shape=[jax.ShapeDtypeStruct((B, D), _f32),
               jax.ShapeDtypeStruct((B, 1), _f32)],
)


def _softmax(x):
    z = x - jnp.max(x, axis=1, keepdims=True)
    e = jnp.exp(z)
    return e / jnp.sum(e, axis=1, keepdims=True)


def _head_body(sums_ref, cnt_ref, gu_ref, gv_ref, cu_ref, cv_ref, rf_ref,
               wp_ref, bp_ref, wm1_ref, bm1_ref, wm2_ref, bm2_ref,
               pred_ref, pu_ref, hsub_ref):
    cnt = jnp.maximum(cnt_ref[...], 1.0)
    pooled = sums_ref[...] / cnt
    hsub = jnp.tanh(jnp.dot(pooled, wp_ref[...], preferred_element_type=_f32)
                    + bp_ref[...])
    gu = gu_ref[...]
    gv = gv_ref[...]
    pu = _softmax(lax.dot_general(gu, cu_ref[...], (((1,), (1,)), ((), ())),
                                  preferred_element_type=_f32))
    pv = _softmax(lax.dot_general(gv, cv_ref[...], (((1,), (1,)), ((), ())),
                                  preferred_element_type=_f32))
    wm1 = wm1_ref[...]
    logits = (jnp.dot(hsub, wm1[0:D], preferred_element_type=_f32)
              + jnp.dot(gu, wm1[D:2 * D], preferred_element_type=_f32)
              + jnp.dot(gv, wm1[2 * D:3 * D], preferred_element_type=_f32)
              + jnp.dot(pu, wm1[3 * D:3 * D + K], preferred_element_type=_f32)
              + jnp.dot(pv, wm1[3 * D + K:3 * D + 2 * K],
                        preferred_element_type=_f32)
              + jnp.dot(rf_ref[...], wm1[3 * D + 2 * K:],
                        preferred_element_type=_f32)
              + bm1_ref[...])
    act = jnp.maximum(logits, 0.0)
    o = jnp.dot(act, wm2_ref[...], preferred_element_type=_f32) + bm2_ref[...]
    pred_ref[...] = 1.0 / (1.0 + jnp.exp(-o))
    pu_ref[...] = pu
    hsub_ref[...] = hsub


_head = pl.pallas_call(
    _head_body,
    out_shape=[jax.ShapeDtypeStruct((B, 1), _f32),
               jax.ShapeDtypeStruct((B, K), _f32),
               jax.ShapeDtypeStruct((B, D), _f32)],
)


def kernel(batch_u, batch_v, global_u_emb, global_v_emb, sub_x,
           sub_edge_index, sub_edge_attr, sub_batch, sub_dist_labels,
           rand_feat, c_u, c_v, dist_label_emb, W_msg0, W_self0, b0,
           W_msg1, W_self1, b1, W_pool, b_pool, W_m1, b_m1, W_m2, b_m2):
    bu = batch_u.astype(_i32)
    bv = batch_v.astype(_i32)
    src = sub_edge_index[0].astype(_i32)
    dst = sub_edge_index[1].astype(_i32)
    attr = sub_edge_attr.reshape(E).astype(_f32)

    pad_e = EP - E
    pad_idx = jnp.arange(pad_e, dtype=_i32) % N
    src_p = jnp.concatenate([src, pad_idx]).reshape(TOTG, G)
    dst_p = jnp.concatenate([dst, pad_idx]).reshape(TOTG, G)
    attr_p = jnp.concatenate([attr, jnp.zeros((pad_e,), _f32)]).reshape(TOTG, G)

    pad_n = NP - N
    x_p = jnp.concatenate([sub_x, jnp.zeros((pad_n, D), _f32)], axis=0)
    lab_p = jnp.concatenate([sub_dist_labels.astype(_i32),
                             jnp.zeros((pad_n,), _i32)]).reshape(NP, 1)
    sb_p = jnp.concatenate([sub_batch.astype(_i32),
                            jnp.full((pad_n,), B, _i32)]).reshape(NP, 1)
    det_p = jnp.concatenate([dist_label_emb, jnp.zeros((16 - L, D), _f32)],
                            axis=0)

    m0, s0 = _prep(x_p, lab_p, det_p, W_msg0, W_self0, b0.reshape(1, D))
    gu, gv = _get_emb_gather()(global_u_emb, global_v_emb, bu, bv)
    edge_agg = _get_edge_agg()
    part0 = edge_agg(m0, src_p, dst_p, attr_p)
    m1, s1 = _comb(part0[0], part0[1], s0, W_msg1, W_self1, b1.reshape(1, D))
    part1 = edge_agg(m1, src_p, dst_p, attr_p)
    sums, cnt = _pool(part1[0], part1[1], s1, sb_p)
    pred, pu, hsub = _head(sums, cnt, gu, gv, c_u, c_v, rand_feat,
                           W_pool, b_pool.reshape(1, D),
                           W_m1, b_m1.reshape(1, K),
                           W_m2, b_m2.reshape(1, 1))
    return pred.reshape(B), pu, hsub, pu


# speedup vs baseline: 2.7125x; 1.0214x over previous
import functools

import jax
import jax.numpy as jnp
from jax import lax
from jax.experimental import pallas as pl
from jax.experimental.pallas import tpu as pltpu
from jax.experimental.pallas import tpu_sc as plsc

B = 1024
D = 128
K = 64
N = 10000
E = 320000
L = 10

NC = 2
NS = 16
NW = NC * NS

G = 128
EP = 327680
TOTG = EP // G
Q0 = 80
Q1 = 80
W = 16
PAIRS_H = W // 2
NP = 10240
RPT = NP // NS
BPW = B // NW

R = 1280
GRID = NP // R

_f32 = jnp.float32
_i32 = jnp.int32


def _mesh():
    return plsc.VectorSubcoreMesh(
        core_axis_name="c", subcore_axis_name="s",
        num_cores=NC, num_subcores=NS)


@functools.cache
def _get_edge_agg():
    @functools.partial(
        pl.kernel,
        mesh=_mesh(),
        out_type=jax.ShapeDtypeStruct((NC, NP, D), _f32),
        scratch_types=[
            pltpu.VMEM_SHARED((NP, D), _f32),
            pltpu.VMEM((W, G), _i32),
            pltpu.VMEM((W, G), _i32),
            pltpu.VMEM((W, G), _f32),
            pltpu.VMEM((G, D), _f32),
            pltpu.VMEM((G, D), _f32),
            pltpu.SemaphoreType.DMA,
            pltpu.SemaphoreType.DMA,
            pltpu.SemaphoreType.DMA,
            pltpu.SemaphoreType.DMA,
        ],
    )
    def _edge_agg(m_hbm, edge_hbm, attr_hbm, part_hbm,
                  acc, srcb, dstb, attrb, rows0, rows1,
                  gsem0, gsem1, ssem0, ssem1):
        c = lax.axis_index("c")
        s = lax.axis_index("s")
        gbase = jnp.where(c == 0, s * Q0, NS * Q0 + s * Q1)
        nwin = jnp.where(c == 0, Q0 // W, Q1 // W)

        def _zr(i, carry):
            for k in range(D // 16):
                rows0[i, pl.ds(k * 16, 16)] = jnp.zeros((16,), _f32)
            return carry
        lax.fori_loop(0, G, _zr, 0)
        for j in range(RPT // G):
            pltpu.sync_copy(rows0, acc.at[pl.ds(s * RPT + j * G, G)])

        plsc.subcore_barrier()

        def _scale(rows, g):
            def _scale16(t, inner):
                avec = attrb[g, pl.ds(t * 16, 16)]
                for j in range(16):
                    a = avec[j]
                    r = t * 16 + j
                    for k in range(D // 16):
                        rows[r, pl.ds(k * 16, 16)] = (
                            rows[r, pl.ds(k * 16, 16)] * a)
                return inner
            lax.fori_loop(0, G // 16, _scale16, 0)

        def _window(win, wcarry):
            wstart = pl.multiple_of(gbase + win * W, W)
            pltpu.sync_copy(edge_hbm.at[0, pl.ds(wstart, W)], srcb)
            pltpu.sync_copy(edge_hbm.at[1, pl.ds(wstart, W)], dstb)
            pltpu.sync_copy(attr_hbm.at[pl.ds(wstart, W)], attrb)
            pltpu.async_copy(m_hbm.at[srcb.at[0]], rows0, gsem0)

            def _pair(p, carry):
                g0 = 2 * p
                g1 = 2 * p + 1
                pltpu.make_async_copy(
                    m_hbm.at[srcb.at[g0]], rows0, gsem0).wait()

                @pl.when(p > 0)
                def _():
                    pltpu.make_async_copy(
                        rows1, acc.at[dstb.at[g0 - 1]], ssem1).wait()
                pltpu.async_copy(m_hbm.at[srcb.at[g1]], rows1, gsem1)
                _scale(rows0, g0)
                pltpu.async_copy(rows0, acc.at[dstb.at[g0]], ssem0, add=True)
                pltpu.make_async_copy(
                    m_hbm.at[srcb.at[g1]], rows1, gsem1).wait()
                pltpu.make_async_copy(rows0, acc.at[dstb.at[g0]], ssem0).wait()

                @pl.when(p < PAIRS_H - 1)
                def _():
                    pltpu.async_copy(m_hbm.at[srcb.at[g1 + 1]], rows0, gsem0)
                _scale(rows1, g1)
                pltpu.async_copy(rows1, acc.at[dstb.at[g1]], ssem1, add=True)
                return carry
            lax.fori_loop(0, PAIRS_H, _pair, 0)
            pltpu.make_async_copy(rows1, acc.at[dstb.at[W - 1]], ssem1).wait()
            return wcarry
        lax.fori_loop(0, nwin, _window, 0)

        plsc.subcore_barrier()
        pltpu.sync_copy(acc.at[pl.ds(s * RPT, RPT)],
                        part_hbm.at[c, pl.ds(s * RPT, RPT)])
    return _edge_agg


@functools.cache
def _get_emb_gather():
    @functools.partial(
        pl.kernel,
        mesh=_mesh(),
        out_type=(jax.ShapeDtypeStruct((B, D), _f32),
                  jax.ShapeDtypeStruct((B, D), _f32)),
        scratch_types=[
            pltpu.VMEM((BPW,), _i32),
            pltpu.VMEM((BPW, D), _f32),
            pltpu.SemaphoreType.DMA,
        ],
    )
    def _emb_gather(ue_hbm, ve_hbm, bu_hbm, bv_hbm, gu_hbm, gv_hbm,
                    idx, buf, sem):
        c = lax.axis_index("c")
        s = lax.axis_index("s")
        base = (c * NS + s) * BPW
        pltpu.sync_copy(bu_hbm.at[pl.ds(base, BPW)], idx)
        pltpu.async_copy(ue_hbm.at[idx], buf, sem).wait()
        pltpu.sync_copy(buf, gu_hbm.at[pl.ds(base, BPW)])
        pltpu.sync_copy(bv_hbm.at[pl.ds(base, BPW)], idx)
        pltpu.async_copy(ve_hbm.at[idx], buf, sem).wait()
        pltpu.sync_copy(buf, gv_hbm.at[pl.ds(base, BPW)])
    return _emb_gather


def _prep_body(x_ref, lab_ref, det_ref, wm_ref, ws_ref, b_ref, m_ref, s_ref):
    lab = lab_ref[...]
    iota = lax.broadcasted_iota(_i32, (R, 16), 1)
    onehot = (lab == iota).astype(_f32)
    de = jnp.dot(onehot, det_ref[...], preferred_element_type=_f32)
    x = x_ref[...]
    wm = wm_ref[...]
    ws = ws_ref[...]
    m_ref[...] = (jnp.dot(x, wm[:D], preferred_element_type=_f32)
                  + jnp.dot(de, wm[D:], preferred_element_type=_f32))
    s_ref[...] = (jnp.dot(x, ws[:D], preferred_element_type=_f32)
                  + jnp.dot(de, ws[D:], preferred_element_type=_f32)
                  + b_ref[...])


_prep = pl.pallas_call(
    _prep_body,
    grid=(GRID,),
    in_specs=[
        pl.BlockSpec((R, D), lambda i: (i, 0)),
        pl.BlockSpec((R, 1), lambda i: (i, 0)),
        pl.BlockSpec((16, D), lambda i: (0, 0)),
        pl.BlockSpec((2 * D, D), lambda i: (0, 0)),
        pl.BlockSpec((2 * D, D), lambda i: (0, 0)),
        pl.BlockSpec((1, D), lambda i: (0, 0)),
    ],
    out_specs=[pl.BlockSpec((R, D), lambda i: (i, 0))] * 2,
    out_shape=[jax.ShapeDtypeStruct((NP, D), _f32)] * 2,
)


def _comb_body(p0_ref, p1_ref, sin_ref, wm_ref, ws_ref, b_ref, m_ref, s_ref):
    h = jnp.maximum(p0_ref[...] + p1_ref[...] + sin_ref[...], 0.0)
    m_ref[...] = jnp.dot(h, wm_ref[...], preferred_element_type=_f32)
    s_ref[...] = jnp.dot(h, ws_ref[...], preferred_element_type=_f32) + b_ref[...]


_comb = pl.pallas_call(
    _comb_body,
    grid=(GRID,),
    in_specs=[
        pl.BlockSpec((R, D), lambda i: (i, 0)),
        pl.BlockSpec((R, D), lambda i: (i, 0)),
        pl.BlockSpec((R, D), lambda i: (i, 0)),
        pl.BlockSpec((D, D), lambda i: (0, 0)),
        pl.BlockSpec((D, D), lambda i: (0, 0)),
        pl.BlockSpec((1, D), lambda i: (0, 0)),
    ],
    out_specs=[pl.BlockSpec((R, D), lambda i: (i, 0))] * 2,
    out_shape=[jax.ShapeDtypeStruct((NP, D), _f32)] * 2,
)


def _pool_body(q0_ref, q1_ref, sin_ref, sb_ref, sums_ref, cnt_ref):
    h = jnp.maximum(q0_ref[...] + q1_ref[...] + sin_ref[...], 0.0)
    sb = sb_ref[...]
    iota = lax.broadcasted_iota(_i32, (R, B), 1)
    onehot = (sb == iota).astype(_f32)
    psum = lax.dot_general(onehot, h, (((0,), (0,)), ((), ())),
                           preferred_element_type=_f32)
    ones = jnp.ones((R, 1), _f32)
    pcnt = lax.dot_general(onehot, ones, (((0,), (0,)), ((), ())),
                           preferred_element_type=_f32)

    @pl.when(pl.program_id(0) == 0)
    def _():
        sums_ref[...] = jnp.zeros_like(sums_ref)
        cnt_ref[...] = jnp.zeros_like(cnt_ref)

    sums_ref[...] += psum
    cnt_ref[...] += pcnt


_pool = pl.pallas_call(
    _pool_body,
    grid=(GRID,),
    in_specs=[
        pl.BlockSpec((R, D), lambda i: (i, 0)),
        pl.BlockSpec((R, D), lambda i: (i, 0)),
        pl.BlockSpec((R, D), lambda i: (i, 0)),
        pl.BlockSpec((R, 1), lambda i: (i, 0)),
    ],
    out_specs=[pl.BlockSpec((B, D), lambda i: (0, 0)),
               pl.BlockSpec((B, 1), lambda i: (0, 0))],
    out_shape=[jax.ShapeDtypeStruct((B, D), _f32),
               jax.ShapeDtypeStruct((B, 1), _f32)],
)


def _softmax(x):
    z = x - jnp.max(x, axis=1, keepdims=True)
    e = jnp.exp(z)
    return e / jnp.sum(e, axis=1, keepdims=True)


def _head_body(sums_ref, cnt_ref, gu_ref, gv_ref, cu_ref, cv_ref, rf_ref,
               wp_ref, bp_ref, wm1_ref, bm1_ref, wm2_ref, bm2_ref,
               pred_ref, pu_ref, hsub_ref):
    cnt = jnp.maximum(cnt_ref[...], 1.0)
    pooled = sums_ref[...] / cnt
    hsub = jnp.tanh(jnp.dot(pooled, wp_ref[...], preferred_element_type=_f32)
                    + bp_ref[...])
    gu = gu_ref[...]
    gv = gv_ref[...]
    pu = _softmax(lax.dot_general(gu, cu_ref[...], (((1,), (1,)), ((), ())),
                                  preferred_element_type=_f32))
    pv = _softmax(lax.dot_general(gv, cv_ref[...], (((1,), (1,)), ((), ())),
                                  preferred_element_type=_f32))
    wm1 = wm1_ref[...]
    logits = (jnp.dot(hsub, wm1[0:D], preferred_element_type=_f32)
              + jnp.dot(gu, wm1[D:2 * D], preferred_element_type=_f32)
              + jnp.dot(gv, wm1[2 * D:3 * D], preferred_element_type=_f32)
              + jnp.dot(pu, wm1[3 * D:3 * D + K], preferred_element_type=_f32)
              + jnp.dot(pv, wm1[3 * D + K:3 * D + 2 * K],
                        preferred_element_type=_f32)
              + jnp.dot(rf_ref[...], wm1[3 * D + 2 * K:],
                        preferred_element_type=_f32)
              + bm1_ref[...])
    act = jnp.maximum(logits, 0.0)
    o = jnp.dot(act, wm2_ref[...], preferred_element_type=_f32) + bm2_ref[...]
    pred_ref[...] = 1.0 / (1.0 + jnp.exp(-o))
    pu_ref[...] = pu
    hsub_ref[...] = hsub


_head = pl.pallas_call(
    _head_body,
    out_shape=[jax.ShapeDtypeStruct((B, 1), _f32),
               jax.ShapeDtypeStruct((B, K), _f32),
               jax.ShapeDtypeStruct((B, D), _f32)],
)


def kernel(batch_u, batch_v, global_u_emb, global_v_emb, sub_x,
           sub_edge_index, sub_edge_attr, sub_batch, sub_dist_labels,
           rand_feat, c_u, c_v, dist_label_emb, W_msg0, W_self0, b0,
           W_msg1, W_self1, b1, W_pool, b_pool, W_m1, b_m1, W_m2, b_m2):
    bu = batch_u.astype(_i32)
    bv = batch_v.astype(_i32)
    attr = sub_edge_attr.reshape(E).astype(_f32)

    pad_e = EP - E
    pad_idx = jnp.arange(pad_e, dtype=_i32) % N
    ei_p = jnp.concatenate(
        [sub_edge_index.astype(_i32),
         jnp.broadcast_to(pad_idx, (2, pad_e))], axis=1).reshape(2, TOTG, G)
    attr_p = jnp.concatenate([attr, jnp.zeros((pad_e,), _f32)]).reshape(TOTG, G)

    pad_n = NP - N
    x_p = jnp.concatenate([sub_x, jnp.zeros((pad_n, D), _f32)], axis=0)
    lab_p = jnp.concatenate([sub_dist_labels.astype(_i32),
                             jnp.zeros((pad_n,), _i32)]).reshape(NP, 1)
    sb_p = jnp.concatenate([sub_batch.astype(_i32),
                            jnp.full((pad_n,), B, _i32)]).reshape(NP, 1)
    det_p = jnp.concatenate([dist_label_emb, jnp.zeros((16 - L, D), _f32)],
                            axis=0)

    m0, s0 = _prep(x_p, lab_p, det_p, W_msg0, W_self0, b0.reshape(1, D))
    gu, gv = _get_emb_gather()(global_u_emb, global_v_emb, bu, bv)
    edge_agg = _get_edge_agg()
    part0 = edge_agg(m0, ei_p, attr_p)
    m1, s1 = _comb(part0[0], part0[1], s0, W_msg1, W_self1, b1.reshape(1, D))
    part1 = edge_agg(m1, ei_p, attr_p)
    sums, cnt = _pool(part1[0], part1[1], s1, sb_p)
    pred, pu, hsub = _head(sums, cnt, gu, gv, c_u, c_v, rand_feat,
                           W_pool, b_pool.reshape(1, D),
                           W_m1, b_m1.reshape(1, K),
                           W_m2, b_m2.reshape(1, 1))
    return pred.reshape(B), pu, hsub, pu


# speedup vs baseline: 2.8224x; 1.0405x over previous
import functools

import jax
import jax.numpy as jnp
from jax import lax
from jax.experimental import pallas as pl
from jax.experimental.pallas import tpu as pltpu
from jax.experimental.pallas import tpu_sc as plsc

B = 1024
D = 128
K = 64
N = 10000
E = 320000
L = 10

NC = 2
NS = 16
NW = NC * NS

G = 128
EP = 327680
TOTG = EP // G
Q0 = 80
Q1 = 80
W = 40
PAIRS_H = W // 2
NP = 10240
RPT = NP // NS
BPW = B // NW

R = 1280
GRID = NP // R

_f32 = jnp.float32
_i32 = jnp.int32


def _mesh():
    return plsc.VectorSubcoreMesh(
        core_axis_name="c", subcore_axis_name="s",
        num_cores=NC, num_subcores=NS)


@functools.cache
def _get_edge_agg():
    @functools.partial(
        pl.kernel,
        mesh=_mesh(),
        out_type=jax.ShapeDtypeStruct((NC, NP, D), _f32),
        scratch_types=[
            pltpu.VMEM_SHARED((NP, D), _f32),
            pltpu.VMEM((W, G), _i32),
            pltpu.VMEM((W, G), _i32),
            pltpu.VMEM((W, G), _f32),
            pltpu.VMEM((G, D), _f32),
            pltpu.VMEM((G, D), _f32),
            pltpu.SemaphoreType.DMA,
            pltpu.SemaphoreType.DMA,
            pltpu.SemaphoreType.DMA,
            pltpu.SemaphoreType.DMA,
        ],
    )
    def _edge_agg(m_hbm, edge_hbm, attr_hbm, part_hbm,
                  acc, srcb, dstb, attrb, rows0, rows1,
                  gsem0, gsem1, ssem0, ssem1):
        c = lax.axis_index("c")
        s = lax.axis_index("s")
        gbase = jnp.where(c == 0, s * Q0, NS * Q0 + s * Q1)
        nwin = jnp.where(c == 0, Q0 // W, Q1 // W)

        def _zr(i, carry):
            for k in range(D // 16):
                rows0[i, pl.ds(k * 16, 16)] = jnp.zeros((16,), _f32)
            return carry
        lax.fori_loop(0, G, _zr, 0)
        for j in range(RPT // G):
            pltpu.sync_copy(rows0, acc.at[pl.ds(s * RPT + j * G, G)])

        plsc.subcore_barrier()

        def _scale(rows, g):
            def _scale16(t, inner):
                avec = attrb[g, pl.ds(t * 16, 16)]
                for j in range(16):
                    a = avec[j]
                    r = t * 16 + j
                    for k in range(D // 16):
                        rows[r, pl.ds(k * 16, 16)] = (
                            rows[r, pl.ds(k * 16, 16)] * a)
                return inner
            lax.fori_loop(0, G // 16, _scale16, 0)

        def _window(win, wcarry):
            wstart = pl.multiple_of(gbase + win * W, W)
            pltpu.sync_copy(edge_hbm.at[0, pl.ds(wstart, W)], srcb)
            pltpu.sync_copy(edge_hbm.at[1, pl.ds(wstart, W)], dstb)
            pltpu.sync_copy(attr_hbm.at[pl.ds(wstart, W)], attrb)
            pltpu.async_copy(m_hbm.at[srcb.at[0]], rows0, gsem0)

            def _pair(p, carry):
                g0 = 2 * p
                g1 = 2 * p + 1
                pltpu.make_async_copy(
                    m_hbm.at[srcb.at[g0]], rows0, gsem0).wait()

                @pl.when(p > 0)
                def _():
                    pltpu.make_async_copy(
                        rows1, acc.at[dstb.at[g0 - 1]], ssem1).wait()
                pltpu.async_copy(m_hbm.at[srcb.at[g1]], rows1, gsem1)
                _scale(rows0, g0)
                pltpu.async_copy(rows0, acc.at[dstb.at[g0]], ssem0, add=True)
                pltpu.make_async_copy(
                    m_hbm.at[srcb.at[g1]], rows1, gsem1).wait()
                pltpu.make_async_copy(rows0, acc.at[dstb.at[g0]], ssem0).wait()

                @pl.when(p < PAIRS_H - 1)
                def _():
                    pltpu.async_copy(m_hbm.at[srcb.at[g1 + 1]], rows0, gsem0)
                _scale(rows1, g1)
                pltpu.async_copy(rows1, acc.at[dstb.at[g1]], ssem1, add=True)
                return carry
            lax.fori_loop(0, PAIRS_H, _pair, 0)
            pltpu.make_async_copy(rows1, acc.at[dstb.at[W - 1]], ssem1).wait()
            return wcarry
        lax.fori_loop(0, nwin, _window, 0)

        plsc.subcore_barrier()
        pltpu.sync_copy(acc.at[pl.ds(s * RPT, RPT)],
                        part_hbm.at[c, pl.ds(s * RPT, RPT)])
    return _edge_agg


@functools.cache
def _get_emb_gather():
    @functools.partial(
        pl.kernel,
        mesh=_mesh(),
        out_type=(jax.ShapeDtypeStruct((B, D), _f32),
                  jax.ShapeDtypeStruct((B, D), _f32)),
        scratch_types=[
            pltpu.VMEM((BPW,), _i32),
            pltpu.VMEM((BPW, D), _f32),
            pltpu.SemaphoreType.DMA,
        ],
    )
    def _emb_gather(ue_hbm, ve_hbm, bu_hbm, bv_hbm, gu_hbm, gv_hbm,
                    idx, buf, sem):
        c = lax.axis_index("c")
        s = lax.axis_index("s")
        base = (c * NS + s) * BPW
        pltpu.sync_copy(bu_hbm.at[pl.ds(base, BPW)], idx)
        pltpu.async_copy(ue_hbm.at[idx], buf, sem).wait()
        pltpu.sync_copy(buf, gu_hbm.at[pl.ds(base, BPW)])
        pltpu.sync_copy(bv_hbm.at[pl.ds(base, BPW)], idx)
        pltpu.async_copy(ve_hbm.at[idx], buf, sem).wait()
        pltpu.sync_copy(buf, gv_hbm.at[pl.ds(base, BPW)])
    return _emb_gather


def _prep_body(x_ref, lab_ref, det_ref, wm_ref, ws_ref, b_ref, m_ref, s_ref):
    lab = lab_ref[...]
    iota = lax.broadcasted_iota(_i32, (R, 16), 1)
    onehot = (lab == iota).astype(_f32)
    de = jnp.dot(onehot, det_ref[...], preferred_element_type=_f32)
    x = x_ref[...]
    wm = wm_ref[...]
    ws = ws_ref[...]
    m_ref[...] = (jnp.dot(x, wm[:D], preferred_element_type=_f32)
                  + jnp.dot(de, wm[D:], preferred_element_type=_f32))
    s_ref[...] = (jnp.dot(x, ws[:D], preferred_element_type=_f32)
                  + jnp.dot(de, ws[D:], preferred_element_type=_f32)
                  + b_ref[...])


_prep = pl.pallas_call(
    _prep_body,
    grid=(GRID,),
    in_specs=[
        pl.BlockSpec((R, D), lambda i: (i, 0)),
        pl.BlockSpec((R, 1), lambda i: (i, 0)),
        pl.BlockSpec((16, D), lambda i: (0, 0)),
        pl.BlockSpec((2 * D, D), lambda i: (0, 0)),
        pl.BlockSpec((2 * D, D), lambda i: (0, 0)),
        pl.BlockSpec((1, D), lambda i: (0, 0)),
    ],
    out_specs=[pl.BlockSpec((R, D), lambda i: (i, 0))] * 2,
    out_shape=[jax.ShapeDtypeStruct((NP, D), _f32)] * 2,
)


def _comb_body(p0_ref, p1_ref, sin_ref, wm_ref, ws_ref, b_ref, m_ref, s_ref):
    h = jnp.maximum(p0_ref[...] + p1_ref[...] + sin_ref[...], 0.0)
    m_ref[...] = jnp.dot(h, wm_ref[...], preferred_element_type=_f32)
    s_ref[...] = jnp.dot(h, ws_ref[...], preferred_element_type=_f32) + b_ref[...]


_comb = pl.pallas_call(
    _comb_body,
    grid=(GRID,),
    in_specs=[
        pl.BlockSpec((R, D), lambda i: (i, 0)),
        pl.BlockSpec((R, D), lambda i: (i, 0)),
        pl.BlockSpec((R, D), lambda i: (i, 0)),
        pl.BlockSpec((D, D), lambda i: (0, 0)),
        pl.BlockSpec((D, D), lambda i: (0, 0)),
        pl.BlockSpec((1, D), lambda i: (0, 0)),
    ],
    out_specs=[pl.BlockSpec((R, D), lambda i: (i, 0))] * 2,
    out_shape=[jax.ShapeDtypeStruct((NP, D), _f32)] * 2,
)


def _pool_body(q0_ref, q1_ref, sin_ref, sb_ref, sums_ref, cnt_ref):
    h = jnp.maximum(q0_ref[...] + q1_ref[...] + sin_ref[...], 0.0)
    sb = sb_ref[...]
    iota = lax.broadcasted_iota(_i32, (R, B), 1)
    onehot = (sb == iota).astype(_f32)
    psum = lax.dot_general(onehot, h, (((0,), (0,)), ((), ())),
                           preferred_element_type=_f32)
    ones = jnp.ones((R, 1), _f32)
    pcnt = lax.dot_general(onehot, ones, (((0,), (0,)), ((), ())),
                           preferred_element_type=_f32)

    @pl.when(pl.program_id(0) == 0)
    def _():
        sums_ref[...] = jnp.zeros_like(sums_ref)
        cnt_ref[...] = jnp.zeros_like(cnt_ref)

    sums_ref[...] += psum
    cnt_ref[...] += pcnt


_pool = pl.pallas_call(
    _pool_body,
    grid=(GRID,),
    in_specs=[
        pl.BlockSpec((R, D), lambda i: (i, 0)),
        pl.BlockSpec((R, D), lambda i: (i, 0)),
        pl.BlockSpec((R, D), lambda i: (i, 0)),
        pl.BlockSpec((R, 1), lambda i: (i, 0)),
    ],
    out_specs=[pl.BlockSpec((B, D), lambda i: (0, 0)),
               pl.BlockSpec((B, 1), lambda i: (0, 0))],
    out_shape=[jax.ShapeDtypeStruct((B, D), _f32),
               jax.ShapeDtypeStruct((B, 1), _f32)],
)


def _softmax(x):
    z = x - jnp.max(x, axis=1, keepdims=True)
    e = jnp.exp(z)
    return e / jnp.sum(e, axis=1, keepdims=True)


def _head_body(sums_ref, cnt_ref, gu_ref, gv_ref, cu_ref, cv_ref, rf_ref,
               wp_ref, bp_ref, wm1_ref, bm1_ref, wm2_ref, bm2_ref,
               pred_ref, pu_ref, hsub_ref):
    cnt = jnp.maximum(cnt_ref[...], 1.0)
    pooled = sums_ref[...] / cnt
    hsub = jnp.tanh(jnp.dot(pooled, wp_ref[...], preferred_element_type=_f32)
                    + bp_ref[...])
    gu = gu_ref[...]
    gv = gv_ref[...]
    pu = _softmax(lax.dot_general(gu, cu_ref[...], (((1,), (1,)), ((), ())),
                                  preferred_element_type=_f32))
    pv = _softmax(lax.dot_general(gv, cv_ref[...], (((1,), (1,)), ((), ())),
                                  preferred_element_type=_f32))
    wm1 = wm1_ref[...]
    logits = (jnp.dot(hsub, wm1[0:D], preferred_element_type=_f32)
              + jnp.dot(gu, wm1[D:2 * D], preferred_element_type=_f32)
              + jnp.dot(gv, wm1[2 * D:3 * D], preferred_element_type=_f32)
              + jnp.dot(pu, wm1[3 * D:3 * D + K], preferred_element_type=_f32)
              + jnp.dot(pv, wm1[3 * D + K:3 * D + 2 * K],
                        preferred_element_type=_f32)
              + jnp.dot(rf_ref[...], wm1[3 * D + 2 * K:],
                        preferred_element_type=_f32)
              + bm1_ref[...])
    act = jnp.maximum(logits, 0.0)
    o = jnp.dot(act, wm2_ref[...], preferred_element_type=_f32) + bm2_ref[...]
    pred_ref[...] = 1.0 / (1.0 + jnp.exp(-o))
    pu_ref[...] = pu
    hsub_ref[...] = hsub


_head = pl.pallas_call(
    _head_body,
    out_shape=[jax.ShapeDtypeStruct((B, 1), _f32),
               jax.ShapeDtypeStruct((B, K), _f32),
               jax.ShapeDtypeStruct((B, D), _f32)],
)


def kernel(batch_u, batch_v, global_u_emb, global_v_emb, sub_x,
           sub_edge_index, sub_edge_attr, sub_batch, sub_dist_labels,
           rand_feat, c_u, c_v, dist_label_emb, W_msg0, W_self0, b0,
           W_msg1, W_self1, b1, W_pool, b_pool, W_m1, b_m1, W_m2, b_m2):
    bu = batch_u.astype(_i32)
    bv = batch_v.astype(_i32)
    attr = sub_edge_attr.reshape(E).astype(_f32)

    pad_e = EP - E
    pad_idx = jnp.arange(pad_e, dtype=_i32) % N
    ei_p = jnp.concatenate(
        [sub_edge_index.astype(_i32),
         jnp.broadcast_to(pad_idx, (2, pad_e))], axis=1).reshape(2, TOTG, G)
    attr_p = jnp.concatenate([attr, jnp.zeros((pad_e,), _f32)]).reshape(TOTG, G)

    pad_n = NP - N
    x_p = jnp.concatenate([sub_x, jnp.zeros((pad_n, D), _f32)], axis=0)
    lab_p = jnp.concatenate([sub_dist_labels.astype(_i32),
                             jnp.zeros((pad_n,), _i32)]).reshape(NP, 1)
    sb_p = jnp.concatenate([sub_batch.astype(_i32),
                            jnp.full((pad_n,), B, _i32)]).reshape(NP, 1)
    det_p = jnp.concatenate([dist_label_emb, jnp.zeros((16 - L, D), _f32)],
                            axis=0)

    m0, s0 = _prep(x_p, lab_p, det_p, W_msg0, W_self0, b0.reshape(1, D))
    gu, gv = _get_emb_gather()(global_u_emb, global_v_emb, bu, bv)
    edge_agg = _get_edge_agg()
    part0 = edge_agg(m0, ei_p, attr_p)
    m1, s1 = _comb(part0[0], part0[1], s0, W_msg1, W_self1, b1.reshape(1, D))
    part1 = edge_agg(m1, ei_p, attr_p)
    sums, cnt = _pool(part1[0], part1[1], s1, sb_p)
    pred, pu, hsub = _head(sums, cnt, gu, gv, c_u, c_v, rand_feat,
                           W_pool, b_pool.reshape(1, D),
                           W_m1, b_m1.reshape(1, K),
                           W_m2, b_m2.reshape(1, 1))
    return pred.reshape(B), pu, hsub, pu


# speedup vs baseline: 2.9070x; 1.0300x over previous
import functools

import jax
import jax.numpy as jnp
from jax import lax
from jax.experimental import pallas as pl
from jax.experimental.pallas import tpu as pltpu
from jax.experimental.pallas import tpu_sc as plsc

B = 1024
D = 128
K = 64
N = 10000
E = 320000
L = 10

NC = 2
NS = 16
NW = NC * NS

G = 128
EP = 327680
TOTG = EP // G
Q0 = 80
Q1 = 80
W = 40
PAIRS_H = W // 2
NP = 10240
RPT = NP // NS
BPW = B // NW

R = 1280
GRID = NP // R

_f32 = jnp.float32
_i32 = jnp.int32


def _mesh():
    return plsc.VectorSubcoreMesh(
        core_axis_name="c", subcore_axis_name="s",
        num_cores=NC, num_subcores=NS)


@functools.cache
def _get_edge_agg():
    @functools.partial(
        pl.kernel,
        mesh=_mesh(),
        out_type=jax.ShapeDtypeStruct((NC, NP, D), _f32),
        scratch_types=[
            pltpu.VMEM_SHARED((NP, D), _f32),
            pltpu.VMEM((W, G), _i32),
            pltpu.VMEM((W, G), _i32),
            pltpu.VMEM((W, G), _f32),
            pltpu.VMEM((G, D), _f32),
            pltpu.VMEM((G, D), _f32),
            pltpu.SemaphoreType.DMA,
            pltpu.SemaphoreType.DMA,
            pltpu.SemaphoreType.DMA,
            pltpu.SemaphoreType.DMA,
        ],
    )
    def _edge_agg(m_hbm, edge_hbm, attr_hbm, part_hbm,
                  acc, srcb, dstb, attrb, rows0, rows1,
                  gsem0, gsem1, ssem0, ssem1):
        c = lax.axis_index("c")
        s = lax.axis_index("s")
        gbase = jnp.where(c == 0, s * Q0, NS * Q0 + s * Q1)
        nwin = jnp.where(c == 0, Q0 // W, Q1 // W)

        def _zr(i, carry):
            for k in range(D // 16):
                rows0[i, pl.ds(k * 16, 16)] = jnp.zeros((16,), _f32)
            return carry
        lax.fori_loop(0, G, _zr, 0)
        for j in range(RPT // G):
            pltpu.sync_copy(rows0, acc.at[pl.ds(s * RPT + j * G, G)])

        plsc.subcore_barrier()

        def _scale(rows, g):
            def _scale16(t, inner):
                avec = attrb[g, pl.ds(t * 16, 16)]
                for j in range(16):
                    a = avec[j]
                    r = t * 16 + j
                    for k in range(D // 16):
                        rows[r, pl.ds(k * 16, 16)] = (
                            rows[r, pl.ds(k * 16, 16)] * a)
                return inner
            lax.fori_loop(0, G // 16, _scale16, 0)

        def _window(win, wcarry):
            wstart = pl.multiple_of(gbase + win * W, W)
            pltpu.sync_copy(edge_hbm.at[0, pl.ds(wstart, W)], srcb)
            pltpu.sync_copy(edge_hbm.at[1, pl.ds(wstart, W)], dstb)
            pltpu.sync_copy(attr_hbm.at[pl.ds(wstart, W)], attrb)
            pltpu.async_copy(m_hbm.at[srcb.at[0]], rows0, gsem0)

            def _pair(p, carry):
                g0 = 2 * p
                g1 = 2 * p + 1
                pltpu.make_async_copy(
                    m_hbm.at[srcb.at[g0]], rows0, gsem0).wait()

                @pl.when(p > 0)
                def _():
                    pltpu.make_async_copy(
                        rows1, acc.at[dstb.at[g0 - 1]], ssem1).wait()
                pltpu.async_copy(m_hbm.at[srcb.at[g1]], rows1, gsem1)
                _scale(rows0, g0)
                pltpu.async_copy(rows0, acc.at[dstb.at[g0]], ssem0, add=True)
                pltpu.make_async_copy(
                    m_hbm.at[srcb.at[g1]], rows1, gsem1).wait()
                pltpu.make_async_copy(rows0, acc.at[dstb.at[g0]], ssem0).wait()

                @pl.when(p < PAIRS_H - 1)
                def _():
                    pltpu.async_copy(m_hbm.at[srcb.at[g1 + 1]], rows0, gsem0)
                _scale(rows1, g1)
                pltpu.async_copy(rows1, acc.at[dstb.at[g1]], ssem1, add=True)
                return carry
            lax.fori_loop(0, PAIRS_H, _pair, 0)
            pltpu.make_async_copy(rows1, acc.at[dstb.at[W - 1]], ssem1).wait()
            return wcarry
        lax.fori_loop(0, nwin, _window, 0)

        plsc.subcore_barrier()
        pltpu.sync_copy(acc.at[pl.ds(s * RPT, RPT)],
                        part_hbm.at[c, pl.ds(s * RPT, RPT)])
    return _edge_agg


@functools.cache
def _get_emb_gather():
    @functools.partial(
        pl.kernel,
        mesh=_mesh(),
        out_type=(jax.ShapeDtypeStruct((B, D), _f32),
                  jax.ShapeDtypeStruct((B, D), _f32)),
        scratch_types=[
            pltpu.VMEM((BPW,), _i32),
            pltpu.VMEM((BPW, D), _f32),
            pltpu.SemaphoreType.DMA,
        ],
    )
    def _emb_gather(ue_hbm, ve_hbm, bu_hbm, bv_hbm, gu_hbm, gv_hbm,
                    idx, buf, sem):
        c = lax.axis_index("c")
        s = lax.axis_index("s")
        base = (c * NS + s) * BPW
        pltpu.sync_copy(bu_hbm.at[pl.ds(base, BPW)], idx)
        pltpu.async_copy(ue_hbm.at[idx], buf, sem).wait()
        pltpu.sync_copy(buf, gu_hbm.at[pl.ds(base, BPW)])
        pltpu.sync_copy(bv_hbm.at[pl.ds(base, BPW)], idx)
        pltpu.async_copy(ve_hbm.at[idx], buf, sem).wait()
        pltpu.sync_copy(buf, gv_hbm.at[pl.ds(base, BPW)])
    return _emb_gather


def _prep_body(x_ref, lab_ref, det_ref, wm_ref, ws_ref, b_ref, m_ref, s_ref):
    lab = lab_ref[...]
    iota = lax.broadcasted_iota(_i32, (R, 16), 1)
    onehot = (lab == iota).astype(_f32)
    de = jnp.dot(onehot, det_ref[...], preferred_element_type=_f32)
    x = x_ref[...]
    wm = wm_ref[...]
    ws = ws_ref[...]
    m_ref[...] = (jnp.dot(x, wm[:D], preferred_element_type=_f32)
                  + jnp.dot(de, wm[D:], preferred_element_type=_f32))
    s_ref[...] = (jnp.dot(x, ws[:D], preferred_element_type=_f32)
                  + jnp.dot(de, ws[D:], preferred_element_type=_f32)
                  + b_ref[...])


_prep = pl.pallas_call(
    _prep_body,
    grid=(GRID,),
    in_specs=[
        pl.BlockSpec((R, D), lambda i: (i, 0)),
        pl.BlockSpec((R, 1), lambda i: (i, 0)),
        pl.BlockSpec((16, D), lambda i: (0, 0)),
        pl.BlockSpec((2 * D, D), lambda i: (0, 0)),
        pl.BlockSpec((2 * D, D), lambda i: (0, 0)),
        pl.BlockSpec((1, D), lambda i: (0, 0)),
    ],
    out_specs=[pl.BlockSpec((R, D), lambda i: (i, 0))] * 2,
    out_shape=[jax.ShapeDtypeStruct((NP, D), _f32)] * 2,
)


def _comb_body(p0_ref, p1_ref, sin_ref, wm_ref, ws_ref, b_ref, m_ref, s_ref):
    h = jnp.maximum(p0_ref[...] + p1_ref[...] + sin_ref[...], 0.0)
    m_ref[...] = jnp.dot(h, wm_ref[...], preferred_element_type=_f32)
    s_ref[...] = jnp.dot(h, ws_ref[...], preferred_element_type=_f32) + b_ref[...]


_comb = pl.pallas_call(
    _comb_body,
    grid=(GRID,),
    in_specs=[
        pl.BlockSpec((R, D), lambda i: (i, 0)),
        pl.BlockSpec((R, D), lambda i: (i, 0)),
        pl.BlockSpec((R, D), lambda i: (i, 0)),
        pl.BlockSpec((D, D), lambda i: (0, 0)),
        pl.BlockSpec((D, D), lambda i: (0, 0)),
        pl.BlockSpec((1, D), lambda i: (0, 0)),
    ],
    out_specs=[pl.BlockSpec((R, D), lambda i: (i, 0))] * 2,
    out_shape=[jax.ShapeDtypeStruct((NP, D), _f32)] * 2,
)


def _pool_body(q0_ref, q1_ref, sin_ref, sb_ref, sums_ref, cnt_ref):
    h = jnp.maximum(q0_ref[...] + q1_ref[...] + sin_ref[...], 0.0)
    sb = sb_ref[...]
    iota = lax.broadcasted_iota(_i32, (R, B), 1)
    onehot = (sb == iota).astype(_f32)
    psum = lax.dot_general(onehot, h, (((0,), (0,)), ((), ())),
                           preferred_element_type=_f32)
    ones = jnp.ones((R, 1), _f32)
    pcnt = lax.dot_general(onehot, ones, (((0,), (0,)), ((), ())),
                           preferred_element_type=_f32)

    @pl.when(pl.program_id(0) == 0)
    def _():
        sums_ref[...] = jnp.zeros_like(sums_ref)
        cnt_ref[...] = jnp.zeros_like(cnt_ref)

    sums_ref[...] += psum
    cnt_ref[...] += pcnt


_pool = pl.pallas_call(
    _pool_body,
    grid=(GRID,),
    in_specs=[
        pl.BlockSpec((R, D), lambda i: (i, 0)),
        pl.BlockSpec((R, D), lambda i: (i, 0)),
        pl.BlockSpec((R, D), lambda i: (i, 0)),
        pl.BlockSpec((R, 1), lambda i: (i, 0)),
    ],
    out_specs=[pl.BlockSpec((B, D), lambda i: (0, 0)),
               pl.BlockSpec((B, 1), lambda i: (0, 0))],
    out_shape=[jax.ShapeDtypeStruct((B, D), _f32),
               jax.ShapeDtypeStruct((B, 1), _f32)],
)


def _softmax(x):
    z = x - jnp.max(x, axis=1, keepdims=True)
    e = jnp.exp(z)
    return e / jnp.sum(e, axis=1, keepdims=True)


def _head_body(sums_ref, cnt_ref, gu_ref, gv_ref, cu_ref, cv_ref, rf_ref,
               wp_ref, bp_ref, wm1_ref, bm1_ref, wm2_ref, bm2_ref,
               pred_ref, pu_ref, hsub_ref):
    cnt = jnp.maximum(cnt_ref[...], 1.0)
    pooled = sums_ref[...] / cnt
    hsub = jnp.tanh(jnp.dot(pooled, wp_ref[...], preferred_element_type=_f32)
                    + bp_ref[...])
    gu = gu_ref[...]
    gv = gv_ref[...]
    pu = _softmax(lax.dot_general(gu, cu_ref[...], (((1,), (1,)), ((), ())),
                                  preferred_element_type=_f32))
    pv = _softmax(lax.dot_general(gv, cv_ref[...], (((1,), (1,)), ((), ())),
                                  preferred_element_type=_f32))
    wm1 = wm1_ref[...]
    logits = (jnp.dot(hsub, wm1[0:D], preferred_element_type=_f32)
              + jnp.dot(gu, wm1[D:2 * D], preferred_element_type=_f32)
              + jnp.dot(gv, wm1[2 * D:3 * D], preferred_element_type=_f32)
              + jnp.dot(pu, wm1[3 * D:3 * D + K], preferred_element_type=_f32)
              + jnp.dot(pv, wm1[3 * D + K:3 * D + 2 * K],
                        preferred_element_type=_f32)
              + jnp.dot(rf_ref[...], wm1[3 * D + 2 * K:],
                        preferred_element_type=_f32)
              + bm1_ref[...])
    act = jnp.maximum(logits, 0.0)
    o = jnp.dot(act, wm2_ref[...], preferred_element_type=_f32) + bm2_ref[...]
    pred_ref[...] = 1.0 / (1.0 + jnp.exp(-o))
    pu_ref[...] = pu
    hsub_ref[...] = hsub


_head = pl.pallas_call(
    _head_body,
    out_shape=[jax.ShapeDtypeStruct((B, 1), _f32),
               jax.ShapeDtypeStruct((B, K), _f32),
               jax.ShapeDtypeStruct((B, D), _f32)],
)


def kernel(batch_u, batch_v, global_u_emb, global_v_emb, sub_x,
           sub_edge_index, sub_edge_attr, sub_batch, sub_dist_labels,
           rand_feat, c_u, c_v, dist_label_emb, W_msg0, W_self0, b0,
           W_msg1, W_self1, b1, W_pool, b_pool, W_m1, b_m1, W_m2, b_m2):
    bu = batch_u.astype(_i32)
    bv = batch_v.astype(_i32)

    pad_e = EP - E
    pad_idx = jnp.arange(pad_e, dtype=_i32) % N
    ei_p = jnp.concatenate(
        [sub_edge_index.astype(_i32),
         jnp.broadcast_to(pad_idx, (2, pad_e))], axis=1).reshape(2, TOTG, G)
    attr_p = jnp.pad(sub_edge_attr.astype(_f32),
                     ((0, pad_e), (0, 0))).reshape(TOTG, G)

    pad_n = NP - N
    x_p = jnp.concatenate([sub_x, jnp.zeros((pad_n, D), _f32)], axis=0)
    lab_p = jnp.concatenate([sub_dist_labels.astype(_i32),
                             jnp.zeros((pad_n,), _i32)]).reshape(NP, 1)
    sb_p = jnp.concatenate([sub_batch.astype(_i32),
                            jnp.full((pad_n,), B, _i32)]).reshape(NP, 1)
    det_p = jnp.concatenate([dist_label_emb, jnp.zeros((16 - L, D), _f32)],
                            axis=0)

    m0, s0 = _prep(x_p, lab_p, det_p, W_msg0, W_self0, b0.reshape(1, D))
    gu, gv = _get_emb_gather()(global_u_emb, global_v_emb, bu, bv)
    edge_agg = _get_edge_agg()
    part0 = edge_agg(m0, ei_p, attr_p)
    m1, s1 = _comb(part0[0], part0[1], s0, W_msg1, W_self1, b1.reshape(1, D))
    part1 = edge_agg(m1, ei_p, attr_p)
    sums, cnt = _pool(part1[0], part1[1], s1, sb_p)
    pred, pu, hsub = _head(sums, cnt, gu, gv, c_u, c_v, rand_feat,
                           W_pool, b_pool.reshape(1, D),
                           W_m1, b_m1.reshape(1, K),
                           W_m2, b_m2.reshape(1, 1))
    return pred.reshape(B), pu, hsub, pu
